# Initial kernel scaffold; baseline (speedup 1.0000x reference)
import jax
import jax.numpy as jnp
from jax.experimental import pallas as pl


def kernel(x):
    raise NotImplementedError("write your pallas kernel here")


# speedup vs baseline: 2.9152x; 2.9152x over previous
import functools
import itertools

import numpy as np
import jax
import jax.numpy as jnp
from jax import lax
from jax.experimental import pallas as pl
from jax.experimental.pallas import tpu as pltpu

_IN_DIM = 10
_NCH = 2
_CONST = 0.8
_NP = 45
_NC = 630


def _pair_combos(n, k):
    x = list(set(frozenset(i) for i in itertools.product(range(n), repeat=k)
                 if len(set(i)) == k))
    return np.array(sorted([sorted(list(i)) for i in x]), dtype=np.int64)


def _disjoint_combos(drcombos):
    combos = []
    dc = [set(i) for i in drcombos.tolist()]
    for idx, i in enumerate(dc):
        for jdx, j in enumerate(dc):
            if not i.intersection(j):
                if [idx, jdx] not in combos and [jdx, idx] not in combos:
                    combos.append([idx, jdx])
    return np.array(sorted(combos), dtype=np.int64)


_DRC = _pair_combos(_IN_DIM, _NCH)
_DRSC = _disjoint_combos(_DRC)

def _extract_mat(comp):
    m = np.zeros((4 * _IN_DIM, _IN_DIM), dtype=np.float32)
    for j in range(_IN_DIM):
        m[4 * j + comp, j] = 1.0
    return m

_MPX = _extract_mat(1)
_MPY = _extract_mat(2)
_MPZ = _extract_mat(3)

_DMAT = np.zeros((_IN_DIM, _NP), dtype=np.float32)
for _c, (_i, _j) in enumerate(_DRC):
    _DMAT[_i, _c] = 1.0
    _DMAT[_j, _c] = -1.0

_PS = np.zeros((_NP, _NC), dtype=np.float32)
for _c, (_i, _j) in enumerate(_DRSC):
    _PS[_i, _c] = 1.0
    _PS[_j, _c] = 1.0

_PAYLOAD = np.zeros((_NC, 8), dtype=np.float32)
_PAYLOAD[:, 0:4] = _DRC[_DRSC].reshape(_NC, 4).astype(np.float32)
_PAYLOAD[:, 4] = _DRSC[:, 0].astype(np.float32)
_PAYLOAD[:, 5] = _DRSC[:, 1].astype(np.float32)

_BB = 512


def _dot(a, b):
    return jnp.dot(a, b, precision=lax.Precision.HIGHEST,
                   preferred_element_type=jnp.float32)


def _asinh(t):
    u = jnp.abs(t)
    r = jnp.log1p(u + u * u / (1.0 + jnp.sqrt(u * u + 1.0)))
    r = jnp.where(u > 1e19, jnp.log(u) + 0.6931471805599453, r)
    return jnp.sign(t) * r


def _body(x_ref, mpx_ref, mpy_ref, mpz_ref, dm_ref, ps_ref, pay_ref, o_ref):
    xr = x_ref[...]
    px = _dot(xr, mpx_ref[...])
    py = _dot(xr, mpy_ref[...])
    pz = _dot(xr, mpz_ref[...])
    pt = jnp.sqrt(px ** 2 + py ** 2)
    eta = _asinh(pz / pt)
    phi = jnp.arctan2(py, px)
    deta = _dot(eta, dm_ref[...])
    pd = _dot(phi, dm_ref[...])
    dphi = (pd + jnp.pi) % (2.0 * jnp.pi) - jnp.pi
    dr = jnp.sqrt(deta ** 2 + dphi ** 2)
    a = jnp.abs(dr - _CONST)
    a = jnp.where(jnp.isnan(a), 3e38, a)
    a = jnp.minimum(a, 3e38)
    m = _dot(a, ps_ref[...])
    mn = jnp.min(m, axis=1, keepdims=True)
    li = lax.broadcasted_iota(jnp.int32, m.shape, 1)
    idx = jnp.min(jnp.where(m == mn, li, 2**30), axis=1, keepdims=True)
    oh = (li == idx).astype(jnp.float32)
    r = _dot(oh, pay_ref[...])
    labs = r[:, 0:4]
    i0 = r[:, 4:5].astype(jnp.int32)
    i1 = r[:, 5:6].astype(jnp.int32)
    l45 = lax.broadcasted_iota(jnp.int32, dr.shape, 1)
    d0 = jnp.sum(jnp.where(l45 == i0, dr, 0.0), axis=1, keepdims=True)
    d1 = jnp.sum(jnp.where(l45 == i1, dr, 0.0), axis=1, keepdims=True)
    o_ref[...] = jnp.concatenate([labs, d0, d1, mn], axis=1)


@jax.jit
def kernel(x):
    b = x.shape[0]
    x2 = x.reshape(b, 4 * _IN_DIM)
    grid = (b // _BB,)
    full = lambda shape: pl.BlockSpec(shape, lambda i: (0, 0))
    out = pl.pallas_call(
        _body,
        grid=grid,
        in_specs=[
            pl.BlockSpec((_BB, 4 * _IN_DIM), lambda i: (i, 0)),
            full(_MPX.shape),
            full(_MPY.shape),
            full(_MPZ.shape),
            full(_DMAT.shape),
            full(_PS.shape),
            full(_PAYLOAD.shape),
        ],
        out_specs=pl.BlockSpec((_BB, 7), lambda i: (i, 0)),
        out_shape=jax.ShapeDtypeStruct((b, 7), jnp.float32),
        compiler_params=pltpu.CompilerParams(
            dimension_semantics=("arbitrary",),
        ),
    )(x2, _MPX, _MPY, _MPZ, _DMAT, _PS, _PAYLOAD)
    return out


# speedup vs baseline: 4.3377x; 1.4880x over previous
import functools
import itertools

import numpy as np
import jax
import jax.numpy as jnp
from jax import lax
from jax.experimental import pallas as pl
from jax.experimental.pallas import tpu as pltpu

_IN_DIM = 10
_NCH = 2
_CONST = 0.8
_NP = 45
_NC = 630


def _pair_combos(n, k):
    x = list(set(frozenset(i) for i in itertools.product(range(n), repeat=k)
                 if len(set(i)) == k))
    return np.array(sorted([sorted(list(i)) for i in x]), dtype=np.int64)


def _disjoint_combos(drcombos):
    combos = []
    dc = [set(i) for i in drcombos.tolist()]
    for idx, i in enumerate(dc):
        for jdx, j in enumerate(dc):
            if not i.intersection(j):
                if [idx, jdx] not in combos and [jdx, idx] not in combos:
                    combos.append([idx, jdx])
    return np.array(sorted(combos), dtype=np.int64)


_DRC = _pair_combos(_IN_DIM, _NCH)
_DRSC = _disjoint_combos(_DRC)

def _extract_mat(comp):
    m = np.zeros((4 * _IN_DIM, _IN_DIM), dtype=np.float32)
    for j in range(_IN_DIM):
        m[4 * j + comp, j] = 1.0
    return m

_MPX = _extract_mat(1)
_MPY = _extract_mat(2)
_MPZ = _extract_mat(3)

_DMAT = np.zeros((_IN_DIM, _NP), dtype=np.float32)
for _c, (_i, _j) in enumerate(_DRC):
    _DMAT[_i, _c] = 1.0
    _DMAT[_j, _c] = -1.0

_PS = np.zeros((_NP, _NC), dtype=np.float32)
for _c, (_i, _j) in enumerate(_DRSC):
    _PS[_i, _c] = 1.0
    _PS[_j, _c] = 1.0

_PAYLOAD = np.zeros((_NC, 8), dtype=np.float32)
_PAYLOAD[:, 0:4] = _DRC[_DRSC].reshape(_NC, 4).astype(np.float32)
_PAYLOAD[:, 4] = _DRSC[:, 0].astype(np.float32)
_PAYLOAD[:, 5] = _DRSC[:, 1].astype(np.float32)

_IP0 = _DRC[:, 0].astype(np.int32).reshape(1, _NP)
_IP1 = _DRC[:, 1].astype(np.int32).reshape(1, _NP)
_IC0 = _DRSC[:, 0].astype(np.int32).reshape(1, _NC)
_IC1 = _DRSC[:, 1].astype(np.int32).reshape(1, _NC)

_BB = 512


def _dot(a, b):
    return jnp.dot(a, b, precision=lax.Precision.HIGHEST,
                   preferred_element_type=jnp.float32)


def _asinh(t):
    u = jnp.abs(t)
    r = jnp.log1p(u + u * u / (1.0 + jnp.sqrt(u * u + 1.0)))
    r = jnp.where(u > 1e19, jnp.log(u) + 0.6931471805599453, r)
    return jnp.sign(t) * r


def _gather_lanes(src, idx_row, bb):
    idx = jnp.broadcast_to(idx_row, (bb, idx_row.shape[1]))
    return jnp.take_along_axis(src, idx, axis=1)


def _body(x_ref, mpx_ref, mpy_ref, mpz_ref, p0_ref, p1_ref, c0_ref, c1_ref,
          pay_ref, o_ref):
    xr = x_ref[...]
    bb = xr.shape[0]
    px = _dot(xr, mpx_ref[...])
    py = _dot(xr, mpy_ref[...])
    pz = _dot(xr, mpz_ref[...])
    pt = jnp.sqrt(px ** 2 + py ** 2)
    eta = _asinh(pz / pt)
    phi = jnp.arctan2(py, px)
    eta0 = _gather_lanes(eta, p0_ref[...], bb)
    eta1 = _gather_lanes(eta, p1_ref[...], bb)
    phi0 = _gather_lanes(phi, p0_ref[...], bb)
    phi1 = _gather_lanes(phi, p1_ref[...], bb)
    deta = eta0 - eta1
    dphi = (phi0 - phi1 + jnp.pi) % (2.0 * jnp.pi) - jnp.pi
    dr = jnp.sqrt(deta ** 2 + dphi ** 2)
    a = jnp.abs(dr - _CONST)
    a = jnp.where(jnp.isnan(a), 3e38, a)
    a = jnp.minimum(a, 3e38)
    m = (_gather_lanes(a, c0_ref[...], bb)
         + _gather_lanes(a, c1_ref[...], bb))
    mn = jnp.min(m, axis=1, keepdims=True)
    li = lax.broadcasted_iota(jnp.int32, m.shape, 1)
    idx = jnp.min(jnp.where(m == mn, li, 2**30), axis=1, keepdims=True)
    oh = (li == idx).astype(jnp.float32)
    r = jnp.dot(oh, pay_ref[...], preferred_element_type=jnp.float32)
    labs = r[:, 0:4]
    i0 = r[:, 4:5].astype(jnp.int32)
    i1 = r[:, 5:6].astype(jnp.int32)
    l45 = lax.broadcasted_iota(jnp.int32, dr.shape, 1)
    d0 = jnp.sum(jnp.where(l45 == i0, dr, 0.0), axis=1, keepdims=True)
    d1 = jnp.sum(jnp.where(l45 == i1, dr, 0.0), axis=1, keepdims=True)
    o_ref[...] = jnp.concatenate([labs, d0, d1, mn], axis=1)


@jax.jit
def kernel(x):
    b = x.shape[0]
    x2 = x.reshape(b, 4 * _IN_DIM)
    grid = (b // _BB,)
    full = lambda shape: pl.BlockSpec(shape, lambda i: (0, 0))
    out = pl.pallas_call(
        _body,
        grid=grid,
        in_specs=[
            pl.BlockSpec((_BB, 4 * _IN_DIM), lambda i: (i, 0)),
            full(_MPX.shape),
            full(_MPY.shape),
            full(_MPZ.shape),
            full(_IP0.shape),
            full(_IP1.shape),
            full(_IC0.shape),
            full(_IC1.shape),
            full(_PAYLOAD.shape),
        ],
        out_specs=pl.BlockSpec((_BB, 7), lambda i: (i, 0)),
        out_shape=jax.ShapeDtypeStruct((b, 7), jnp.float32),
        compiler_params=pltpu.CompilerParams(
            dimension_semantics=("arbitrary",),
        ),
    )(x2, _MPX, _MPY, _MPZ, _IP0, _IP1, _IC0, _IC1, _PAYLOAD)
    return out


# speedup vs baseline: 4.3795x; 1.0096x over previous
import functools
import itertools

import numpy as np
import jax
import jax.numpy as jnp
from jax import lax
from jax.experimental import pallas as pl
from jax.experimental.pallas import tpu as pltpu

_IN_DIM = 10
_NCH = 2
_CONST = 0.8
_NP = 45
_NC = 630


def _pair_combos(n, k):
    x = list(set(frozenset(i) for i in itertools.product(range(n), repeat=k)
                 if len(set(i)) == k))
    return np.array(sorted([sorted(list(i)) for i in x]), dtype=np.int64)


def _disjoint_combos(drcombos):
    combos = []
    dc = [set(i) for i in drcombos.tolist()]
    for idx, i in enumerate(dc):
        for jdx, j in enumerate(dc):
            if not i.intersection(j):
                if [idx, jdx] not in combos and [jdx, idx] not in combos:
                    combos.append([idx, jdx])
    return np.array(sorted(combos), dtype=np.int64)


_DRC = _pair_combos(_IN_DIM, _NCH)
_DRSC = _disjoint_combos(_DRC)

_MPXYZ = np.zeros((4 * _IN_DIM, 48), dtype=np.float32)
for _j in range(_IN_DIM):
    for _k, _comp in enumerate((1, 2, 3)):
        _MPXYZ[4 * _j + _comp, 16 * _k + _j] = 1.0

_DMAT = np.zeros((_IN_DIM, _NP), dtype=np.float32)
for _c, (_i, _j) in enumerate(_DRC):
    _DMAT[_i, _c] = 1.0
    _DMAT[_j, _c] = -1.0

_PS = np.zeros((_NP, _NC), dtype=np.float32)
for _c, (_i, _j) in enumerate(_DRSC):
    _PS[_i, _c] = 1.0
    _PS[_j, _c] = 1.0

_PAYLOAD = np.zeros((_NC, 8), dtype=np.float32)
_PAYLOAD[:, 0:4] = _DRC[_DRSC].reshape(_NC, 4).astype(np.float32)
_PAYLOAD[:, 4] = _DRSC[:, 0].astype(np.float32)
_PAYLOAD[:, 5] = _DRSC[:, 1].astype(np.float32)

_IP0 = _DRC[:, 0].astype(np.int32).reshape(1, _NP)
_IP1 = _DRC[:, 1].astype(np.int32).reshape(1, _NP)
_IC0 = _DRSC[:, 0].astype(np.int32).reshape(1, _NC)
_IC1 = _DRSC[:, 1].astype(np.int32).reshape(1, _NC)

_LIDX = np.arange(_NC, dtype=np.float32).reshape(1, _NC)

_BB = 512


def _dot(a, b):
    return jnp.dot(a, b, precision=lax.Precision.HIGHEST,
                   preferred_element_type=jnp.float32)


def _asinh(t):
    u = jnp.abs(t)
    r = jnp.log1p(u + u * u / (1.0 + jnp.sqrt(u * u + 1.0)))
    r = jnp.where(u > 1e19, jnp.log(u) + 0.6931471805599453, r)
    return jnp.sign(t) * r


def _gather_lanes(src, idx_row, bb):
    idx = jnp.broadcast_to(idx_row, (bb, idx_row.shape[1]))
    return jnp.take_along_axis(src, idx, axis=1)


def _body(x_ref, mpxyz_ref, p0_ref, p1_ref, c0_ref, c1_ref, li_ref,
          pay_ref, o_ref):
    xr = x_ref[...]
    bb = xr.shape[0]
    pxyz = jnp.transpose(_dot(xr, mpxyz_ref[...]))
    px = pxyz[0:16]
    py = pxyz[16:32]
    pz = pxyz[32:48]
    pt = jnp.sqrt(px ** 2 + py ** 2)
    eta = jnp.transpose(_asinh(pz / pt))
    phi = jnp.transpose(jnp.arctan2(py, px))
    eta0 = _gather_lanes(eta, p0_ref[...], bb)
    eta1 = _gather_lanes(eta, p1_ref[...], bb)
    phi0 = _gather_lanes(phi, p0_ref[...], bb)
    phi1 = _gather_lanes(phi, p1_ref[...], bb)
    deta = eta0 - eta1
    dphi = (phi0 - phi1 + jnp.pi) % (2.0 * jnp.pi) - jnp.pi
    dr = jnp.sqrt(deta ** 2 + dphi ** 2)
    a = jnp.abs(dr - _CONST)
    a = jnp.where(jnp.isnan(a), 3e38, a)
    a = jnp.minimum(a, 3e38)
    m = (_gather_lanes(a, c0_ref[...], bb)
         + _gather_lanes(a, c1_ref[...], bb))
    mn = jnp.min(m, axis=1, keepdims=True)
    li = jnp.broadcast_to(li_ref[...], m.shape)
    idx = jnp.min(jnp.where(m == mn, li, 1e9), axis=1, keepdims=True)
    oh = (li == idx).astype(jnp.float32)
    r = jnp.dot(oh, pay_ref[...], preferred_element_type=jnp.float32)
    labs = r[:, 0:4]
    i0 = r[:, 4:5].astype(jnp.int32)
    i1 = r[:, 5:6].astype(jnp.int32)
    d0 = jnp.take_along_axis(dr, i0, axis=1)
    d1 = jnp.take_along_axis(dr, i1, axis=1)
    o_ref[...] = jnp.concatenate([labs, d0, d1, mn], axis=1)


@jax.jit
def kernel(x):
    b = x.shape[0]
    x2 = x.reshape(b, 4 * _IN_DIM)
    grid = (b // _BB,)
    full = lambda shape: pl.BlockSpec(shape, lambda i: (0, 0))
    out = pl.pallas_call(
        _body,
        grid=grid,
        in_specs=[
            pl.BlockSpec((_BB, 4 * _IN_DIM), lambda i: (i, 0)),
            full(_MPXYZ.shape),
            full(_IP0.shape),
            full(_IP1.shape),
            full(_IC0.shape),
            full(_IC1.shape),
            full(_LIDX.shape),
            full(_PAYLOAD.shape),
        ],
        out_specs=pl.BlockSpec((_BB, 7), lambda i: (i, 0)),
        out_shape=jax.ShapeDtypeStruct((b, 7), jnp.float32),
        compiler_params=pltpu.CompilerParams(
            dimension_semantics=("arbitrary",),
        ),
    )(x2, _MPXYZ, _IP0, _IP1, _IC0, _IC1, _LIDX, _PAYLOAD)
    return out


# speedup vs baseline: 4.5790x; 1.0456x over previous
import functools
import itertools

import numpy as np
import jax
import jax.numpy as jnp
from jax import lax
from jax.experimental import pallas as pl
from jax.experimental.pallas import tpu as pltpu

_IN_DIM = 10
_NCH = 2
_CONST = 0.8
_NP = 45
_NC = 630


def _pair_combos(n, k):
    x = list(set(frozenset(i) for i in itertools.product(range(n), repeat=k)
                 if len(set(i)) == k))
    return np.array(sorted([sorted(list(i)) for i in x]), dtype=np.int64)


def _disjoint_combos(drcombos):
    combos = []
    dc = [set(i) for i in drcombos.tolist()]
    for idx, i in enumerate(dc):
        for jdx, j in enumerate(dc):
            if not i.intersection(j):
                if [idx, jdx] not in combos and [jdx, idx] not in combos:
                    combos.append([idx, jdx])
    return np.array(sorted(combos), dtype=np.int64)


_DRC = _pair_combos(_IN_DIM, _NCH)
_DRSC = _disjoint_combos(_DRC)

_MPXYZ = np.zeros((4 * _IN_DIM, 48), dtype=np.float32)
for _j in range(_IN_DIM):
    for _k, _comp in enumerate((1, 2, 3)):
        _MPXYZ[4 * _j + _comp, 16 * _k + _j] = 1.0

_DMAT = np.zeros((_IN_DIM, _NP), dtype=np.float32)
for _c, (_i, _j) in enumerate(_DRC):
    _DMAT[_i, _c] = 1.0
    _DMAT[_j, _c] = -1.0

_PS = np.zeros((_NP, _NC), dtype=np.float32)
for _c, (_i, _j) in enumerate(_DRSC):
    _PS[_i, _c] = 1.0
    _PS[_j, _c] = 1.0

_PAYLOAD = np.zeros((_NC, 8), dtype=np.float32)
_PAYLOAD[:, 0:4] = _DRC[_DRSC].reshape(_NC, 4).astype(np.float32)
_PAYLOAD[:, 4] = _DRSC[:, 0].astype(np.float32)
_PAYLOAD[:, 5] = _DRSC[:, 1].astype(np.float32)

_IP0 = _DRC[:, 0].astype(np.int32).reshape(1, _NP)
_IP1 = _DRC[:, 1].astype(np.int32).reshape(1, _NP)
_IC0 = _DRSC[:, 0].astype(np.int32).reshape(1, _NC)
_IC1 = _DRSC[:, 1].astype(np.int32).reshape(1, _NC)

_LIDX = np.arange(_NC, dtype=np.float32).reshape(1, _NC)

_BB = 1024


def _dot(a, b):
    return jnp.dot(a, b, precision=lax.Precision.HIGHEST,
                   preferred_element_type=jnp.float32)


def _asinh(t):
    u = jnp.abs(t)
    r = jnp.log1p(u + u * u / (1.0 + jnp.sqrt(u * u + 1.0)))
    r = jnp.where(u > 1e19, jnp.log(u) + 0.6931471805599453, r)
    return jnp.sign(t) * r


def _gather_lanes(src, idx_row, bb):
    idx = jnp.broadcast_to(idx_row, (bb, idx_row.shape[1]))
    return jnp.take_along_axis(src, idx, axis=1)


def _body(x_ref, mpxyz_ref, p0_ref, p1_ref, c0_ref, c1_ref, li_ref,
          pay_ref, o_ref):
    xr = x_ref[...]
    bb = xr.shape[0]
    pxyz = jnp.transpose(_dot(xr, mpxyz_ref[...]))
    px = pxyz[0:16]
    py = pxyz[16:32]
    pz = pxyz[32:48]
    pt = jnp.sqrt(px ** 2 + py ** 2)
    eta = jnp.transpose(_asinh(pz / pt))
    phi = jnp.transpose(jnp.arctan2(py, px))
    eta0 = _gather_lanes(eta, p0_ref[...], bb)
    eta1 = _gather_lanes(eta, p1_ref[...], bb)
    phi0 = _gather_lanes(phi, p0_ref[...], bb)
    phi1 = _gather_lanes(phi, p1_ref[...], bb)
    deta = eta0 - eta1
    dphi = (phi0 - phi1 + jnp.pi) % (2.0 * jnp.pi) - jnp.pi
    dr = jnp.sqrt(deta ** 2 + dphi ** 2)
    a = jnp.abs(dr - _CONST)
    a = jnp.where(jnp.isnan(a), 3e38, a)
    a = jnp.minimum(a, 3e38)
    m = (_gather_lanes(a, c0_ref[...], bb)
         + _gather_lanes(a, c1_ref[...], bb))
    mn = jnp.min(m, axis=1, keepdims=True)
    li = jnp.broadcast_to(li_ref[...], m.shape)
    idx = jnp.min(jnp.where(m == mn, li, 1e9), axis=1, keepdims=True)
    oh = (li == idx).astype(jnp.float32)
    r = jnp.dot(oh, pay_ref[...], preferred_element_type=jnp.float32)
    labs = r[:, 0:4]
    i0 = r[:, 4:5].astype(jnp.int32)
    i1 = r[:, 5:6].astype(jnp.int32)
    d0 = jnp.take_along_axis(dr, i0, axis=1)
    d1 = jnp.take_along_axis(dr, i1, axis=1)
    o_ref[...] = jnp.concatenate([labs, d0, d1, mn], axis=1)


@jax.jit
def kernel(x):
    b = x.shape[0]
    x2 = x.reshape(b, 4 * _IN_DIM)
    grid = (b // _BB,)
    full = lambda shape: pl.BlockSpec(shape, lambda i: (0, 0))
    out = pl.pallas_call(
        _body,
        grid=grid,
        in_specs=[
            pl.BlockSpec((_BB, 4 * _IN_DIM), lambda i: (i, 0)),
            full(_MPXYZ.shape),
            full(_IP0.shape),
            full(_IP1.shape),
            full(_IC0.shape),
            full(_IC1.shape),
            full(_LIDX.shape),
            full(_PAYLOAD.shape),
        ],
        out_specs=pl.BlockSpec((_BB, 7), lambda i: (i, 0)),
        out_shape=jax.ShapeDtypeStruct((b, 7), jnp.float32),
        compiler_params=pltpu.CompilerParams(
            dimension_semantics=("arbitrary",),
        ),
    )(x2, _MPXYZ, _IP0, _IP1, _IC0, _IC1, _LIDX, _PAYLOAD)
    return out
